# baseline (speedup 1.0000x reference)
import functools

import jax
import jax.numpy as jnp
from jax.experimental import pallas as pl
from jax.experimental.pallas import tpu as pltpu

_TEMPERATURE = 0.07
_TOP_K = 2

_BM = 2048
_BN = 2048
_BK = 256


def _router_body(bn, n_blocks, k_blocks,
                 x_ref, w_ref, be_ref, wr_ref, br_ref,
                 coeff_ref, tv_ref, ti_ref,
                 acc_a, acc_b, norms_ref):
    m = pl.program_id(0)
    n = pl.program_id(1)
    k = pl.program_id(2)
    c = m * n_blocks + n
    cells = pl.num_programs(0) * n_blocks
    chunk = bn // k_blocks

    def tail_write():
        norms = norms_ref[...] + br_ref[...]
        nd = norms.shape[1]
        mean = jnp.mean(norms, axis=1, keepdims=True)
        var = jnp.sum((norms - mean) ** 2, axis=1, keepdims=True) / (nd - 1)
        std = jnp.sqrt(var) + 1e-6
        z = (norms - mean) / (std * _TEMPERATURE)
        z = z - jnp.max(z, axis=1, keepdims=True)
        e = jnp.exp(z)
        coeff = e / jnp.sum(e, axis=1, keepdims=True)
        coeff_ref[...] = coeff

        lane = jax.lax.broadcasted_iota(jnp.int32, coeff.shape, 1)
        v0 = jnp.max(coeff, axis=1, keepdims=True)
        i0 = jnp.min(jnp.where(coeff == v0, lane, nd), axis=1, keepdims=True)
        masked = jnp.where(lane == i0, -1.0, coeff)
        v1 = jnp.max(masked, axis=1, keepdims=True)
        i1 = jnp.min(jnp.where(masked == v1, lane, nd), axis=1, keepdims=True)
        tv_ref[...] = jnp.concatenate([v0, v1], axis=1)
        ti_ref[...] = jnp.concatenate([i0, i1], axis=1)

    def step(cur, prev):
        @pl.when(k == 0)
        def _():
            cur[...] = jnp.dot(x_ref[...], w_ref[...],
                               preferred_element_type=jnp.float32)

        @pl.when(k > 0)
        def _():
            cur[...] += jnp.dot(x_ref[...], w_ref[...],
                                preferred_element_type=jnp.float32)

        @pl.when(c > 0)
        def _():
            pn = (c - 1) % n_blocks
            base = pn * bn + k * chunk
            seg = prev[:, pl.ds(k * chunk, chunk)]
            feat = jax.nn.gelu(seg + be_ref[:, pl.ds(base, chunk)])
            part = jnp.dot(feat, wr_ref[pl.ds(base, chunk), :],
                           preferred_element_type=jnp.float32)

            @pl.when((pn == 0) & (k == 0))
            def _():
                norms_ref[...] = part

            @pl.when((pn > 0) | (k > 0))
            def _():
                norms_ref[...] += part

        @pl.when((c > 0) & ((c - 1) % n_blocks == n_blocks - 1)
                 & (k == k_blocks - 1))
        def _():
            tail_write()

        @pl.when((c == cells - 1) & (k == k_blocks - 1))
        def _():
            feat = jax.nn.gelu(cur[...] + be_ref[:, pl.ds(n * bn, bn)])
            part = jnp.dot(feat, wr_ref[pl.ds(n * bn, bn), :],
                           preferred_element_type=jnp.float32)
            norms_ref[...] += part
            tail_write()

    @pl.when(c % 2 == 0)
    def _():
        step(acc_a, acc_b)

    @pl.when(c % 2 == 1)
    def _():
        step(acc_b, acc_a)


@jax.jit
def kernel(images, W_enc, b_enc, W_router, b_router):
    B, D = images.shape
    ND = W_router.shape[1]
    bm, bn, bk = min(_BM, B), min(_BN, D), min(_BK, D)
    m_blocks, n_blocks, k_blocks = B // bm, D // bn, D // bk

    def out_idx(m, n, k):
        c = m * n_blocks + n
        return (jnp.maximum(c - 1, 0) // n_blocks, 0)

    body = functools.partial(_router_body, bn, n_blocks, k_blocks)
    coeff, tv, ti = pl.pallas_call(
        body,
        grid=(m_blocks, n_blocks, k_blocks),
        in_specs=[
            pl.BlockSpec((bm, bk), lambda m, n, k: (m, k)),
            pl.BlockSpec((bk, bn), lambda m, n, k: (k, n)),
            pl.BlockSpec((1, D), lambda m, n, k: (0, 0)),
            pl.BlockSpec((D, ND), lambda m, n, k: (0, 0)),
            pl.BlockSpec((1, ND), lambda m, n, k: (0, 0)),
        ],
        out_specs=[
            pl.BlockSpec((bm, ND), out_idx),
            pl.BlockSpec((bm, _TOP_K), out_idx),
            pl.BlockSpec((bm, _TOP_K), out_idx),
        ],
        out_shape=[
            jax.ShapeDtypeStruct((B, ND), jnp.float32),
            jax.ShapeDtypeStruct((B, _TOP_K), jnp.float32),
            jax.ShapeDtypeStruct((B, _TOP_K), jnp.int32),
        ],
        scratch_shapes=[
            pltpu.VMEM((bm, bn), jnp.float32),
            pltpu.VMEM((bm, bn), jnp.float32),
            pltpu.VMEM((bm, ND), jnp.float32),
        ],
        compiler_params=pltpu.CompilerParams(
            dimension_semantics=("arbitrary", "arbitrary", "arbitrary"),
        ),
    )(images, W_enc, b_enc.reshape(1, D), W_router, b_router.reshape(1, ND))
    return (coeff, tv, ti)
